# baseline (speedup 1.0000x reference)
import functools

import jax
import jax.numpy as jnp
from jax.experimental import pallas as pl
from jax.experimental.pallas import tpu as pltpu
from jax.experimental.pallas import tpu_sc as plsc

_ROW_IDS = (5, 8, 7, 16, 256, 123)
_NUM_ROWS = len(_ROW_IDS)


def kernel(inputs):
    _, d = inputs.shape

    mesh = plsc.ScalarSubcoreMesh(axis_name="c", num_cores=1)

    @functools.partial(
        pl.kernel,
        mesh=mesh,
        out_type=jax.ShapeDtypeStruct((_NUM_ROWS, d), jnp.float32),
        scratch_types=[pltpu.SemaphoreType.DMA],
    )
    def gather_rows(table_hbm, out_hbm, sem):
        copies = [
            pltpu.make_async_copy(
                table_hbm.at[pl.ds(row, 1)], out_hbm.at[pl.ds(i, 1)], sem
            )
            for i, row in enumerate(_ROW_IDS)
        ]
        for c in copies:
            c.start()
        pltpu.make_async_copy(table_hbm.at[pl.ds(0, _NUM_ROWS)], out_hbm, sem).wait()

    return gather_rows(inputs)
